# Initial kernel scaffold; baseline (speedup 1.0000x reference)
import jax
import jax.numpy as jnp
from jax.experimental import pallas as pl


def kernel(boxes, scores):
    raise NotImplementedError("write your pallas kernel here")


# speedup vs baseline: 532.8513x; 532.8513x over previous
import jax
import jax.numpy as jnp
from jax import lax
from jax.experimental import pallas as pl
from jax.experimental.pallas import tpu as pltpu
from jax.experimental.pallas import tpu_sc as plsc

_SCORE_THRESH = 0.05
_NMS_THRESH = 0.5
_DET = 100
_N = 5000
_L = 16
_TILES = 16
_PER_TILE = 320
_NVEC = _PER_TILE // _L
_NPAD = _TILES * _PER_TILE

def _shuf(v, lanes, sh):
    return v.at[lanes ^ sh].get(mode="promise_in_bounds")


def _tree_argmax(val, idx, lanes):
    for sh in (8, 4, 2, 1):
        pv = _shuf(val, lanes, sh)
        pi = _shuf(idx, lanes, sh)
        gt = pv > val
        eq = pv == val
        idx = jnp.where(gt, pi, jnp.where(eq, jnp.minimum(idx, pi), idx))
        val = jnp.where(gt, pv, val)
    return val, idx


def _tree_min(v, lanes):
    for sh in (8, 4, 2, 1):
        v = jnp.minimum(v, _shuf(v, lanes, sh))
    return v


def _nms_body(x1h, y1h, x2h, y2h, sh, outh,
              x1v, y1v, x2v, y2v, sv, alive, taken, row_v, rb_v, out_v, shared):
    c = lax.axis_index("c")
    s = lax.axis_index("s")
    base = s * _PER_TILE

    pltpu.sync_copy(x1h.at[pl.ds(base, _PER_TILE)], x1v)
    pltpu.sync_copy(y1h.at[pl.ds(base, _PER_TILE)], y1v)
    pltpu.sync_copy(x2h.at[pl.ds(base, _PER_TILE)], x2v)
    pltpu.sync_copy(y2h.at[pl.ds(base, _PER_TILE)], y2v)
    pltpu.sync_copy(sh.at[pl.ds(base, _PER_TILE)], sv)

    ones = jnp.ones((_L,), jnp.float32)
    zeros = jnp.zeros((_L,), jnp.float32)
    lanes = lax.iota(jnp.int32, _L)

    def init_j(j, carry):
        alive[pl.ds(j * _L, _L)] = ones
        taken[pl.ds(j * _L, _L)] = zeros
        return carry

    lax.fori_loop(0, _NVEC, init_j, 0)

    def round_body(k, carry):
        def amax_j(j, st):
            bestv, besti = st
            sl = pl.ds(j * _L, _L)
            svv = sv[sl]
            av = alive[sl] > 0.5
            tv = taken[sl] > 0.5
            key = jnp.where(av & (svv > _SCORE_THRESH), svv + 2.0,
                            jnp.where(tv, jnp.float32(-1e9), svv))
            gidx = base + j * _L + lanes
            upd = key > bestv
            return (jnp.where(upd, key, bestv), jnp.where(upd, gidx, besti))

        bestv, besti = lax.fori_loop(
            0, _NVEC, amax_j,
            (jnp.full((_L,), -3e9, jnp.float32), jnp.zeros((_L,), jnp.int32)))
        kmaxv, gminv = _tree_argmax(bestv, besti, lanes)
        lidxv = jnp.clip(gminv - base, 0, _PER_TILE - 1)
        wx1 = plsc.load_gather(x1v, [lidxv])
        wy1 = plsc.load_gather(y1v, [lidxv])
        wx2 = plsc.load_gather(x2v, [lidxv])
        wy2 = plsc.load_gather(y2v, [lidxv])
        wsc = plsc.load_gather(sv, [lidxv])
        row = jnp.where(lanes == 0, kmaxv,
              jnp.where(lanes == 1, gminv.astype(jnp.float32),
              jnp.where(lanes == 2, wx1,
              jnp.where(lanes == 3, wy1,
              jnp.where(lanes == 4, wx2,
              jnp.where(lanes == 5, wy2,
              jnp.where(lanes == 6, wsc, zeros)))))))
        row_v[...] = row
        pltpu.sync_copy(row_v, shared.at[pl.ds(s * _L, _L)])
        plsc.subcore_barrier()
        pltpu.sync_copy(shared, rb_v)
        plsc.subcore_barrier()

        keys = plsc.load_gather(rb_v, [lanes * _L])
        gidxs = plsc.load_gather(rb_v, [lanes * _L + 1])
        kmgv, gmgv = _tree_argmax(keys, gidxs, lanes)
        rstarv = _tree_min(
            jnp.where((keys == kmgv) & (gidxs == gmgv), lanes,
                      jnp.int32(_TILES + 1)), lanes)
        wbv = rstarv * _L

        def bcast(off):
            return plsc.load_gather(rb_v, [wbv + off])

        wx1g = bcast(2)
        wy1g = bcast(3)
        wx2g = bcast(4)
        wy2g = bcast(5)
        wsg = bcast(6)
        keptv = kmgv > 2.0
        areaw = (wx2g - wx1g) * (wy2g - wy1g)

        def supp_j(j, st):
            sl = pl.ds(j * _L, _L)
            ax1 = x1v[sl]
            ay1 = y1v[sl]
            ax2 = x2v[sl]
            ay2 = y2v[sl]
            ltx = jnp.maximum(wx1g, ax1)
            lty = jnp.maximum(wy1g, ay1)
            rbx = jnp.minimum(wx2g, ax2)
            rby = jnp.minimum(wy2g, ay2)
            inter = jnp.maximum(rbx - ltx, 0.0) * jnp.maximum(rby - lty, 0.0)
            areaa = (ax2 - ax1) * (ay2 - ay1)
            iou = inter / (areaa + areaw - inter + 1e-9)
            alive[sl] = jnp.where(iou > _NMS_THRESH, 0.0, alive[sl])
            return st

        lax.fori_loop(0, _NVEC, supp_j, 0)

        ilw = gmgv.astype(jnp.int32) - base
        owner = (ilw >= 0) & (ilw < _PER_TILE) & (lanes == 0)
        ilwc = jnp.clip(ilw, 0, _PER_TILE - 1)
        plsc.store_scatter(taken, [ilwc], ones, mask=owner)
        plsc.store_scatter(alive, [ilwc], zeros, mask=owner)

        @pl.when((c == 0) & (s == 0))
        def _emit():
            sout = jnp.where(keptv, wsg, jnp.full((_L,), -1.0, jnp.float32))
            orow = jnp.where(lanes == 0, wx1g,
                   jnp.where(lanes == 1, wy1g,
                   jnp.where(lanes == 2, wx2g,
                   jnp.where(lanes == 3, wy2g,
                   jnp.where(lanes == 4, sout, zeros)))))
            plsc.store_scatter(out_v, [k * _L + lanes], orow)

        return carry

    lax.fori_loop(0, _DET, round_body, 0)

    @pl.when((c == 0) & (s == 0))
    def _flush():
        pltpu.sync_copy(out_v, outh)


_nms_call = pl.kernel(
    _nms_body,
    out_type=jax.ShapeDtypeStruct((_DET * _L,), jnp.float32),
    mesh=plsc.VectorSubcoreMesh(core_axis_name="c", subcore_axis_name="s"),
    compiler_params=pltpu.CompilerParams(needs_layout_passes=False),
    scratch_types=[
        pltpu.VMEM((_PER_TILE,), jnp.float32),
        pltpu.VMEM((_PER_TILE,), jnp.float32),
        pltpu.VMEM((_PER_TILE,), jnp.float32),
        pltpu.VMEM((_PER_TILE,), jnp.float32),
        pltpu.VMEM((_PER_TILE,), jnp.float32),
        pltpu.VMEM((_PER_TILE,), jnp.float32),
        pltpu.VMEM((_PER_TILE,), jnp.float32),
        pltpu.VMEM((_L,), jnp.float32),
        pltpu.VMEM((_TILES * _L,), jnp.float32),
        pltpu.VMEM((_DET * _L,), jnp.float32),
        pltpu.VMEM_SHARED((_TILES * _L,), jnp.float32),
    ],
)


@jax.jit
def kernel(boxes, scores):
    pad = _NPAD - _N
    x1 = jnp.pad(boxes[:, 0], (0, pad))
    y1 = jnp.pad(boxes[:, 1], (0, pad))
    x2 = jnp.pad(boxes[:, 2], (0, pad))
    y2 = jnp.pad(boxes[:, 3], (0, pad))
    sp = jnp.pad(scores, (0, pad), constant_values=-1.0)
    out = _nms_call(x1, y1, x2, y2, sp)
    return out.reshape(_DET, _L)[:, :5]


# speedup vs baseline: 615.4770x; 1.1551x over previous
import jax
import jax.numpy as jnp
from jax import lax
from jax.experimental import pallas as pl
from jax.experimental.pallas import tpu as pltpu
from jax.experimental.pallas import tpu_sc as plsc

_SCORE_THRESH = 0.05
_NMS_THRESH = 0.5
_DET = 100
_N = 5000
_L = 16
_TILES = 16
_PER_TILE = 320
_NVEC = _PER_TILE // _L
_NPAD = _TILES * _PER_TILE
_XCH = _TILES * _L


def _shuf(v, lanes, sh):
    return v.at[lanes ^ sh].get(mode="promise_in_bounds")


def _tree_argmax(val, idx, lanes):
    for sh in (8, 4, 2, 1):
        pv = _shuf(val, lanes, sh)
        pi = _shuf(idx, lanes, sh)
        gt = pv > val
        eq = pv == val
        idx = jnp.where(gt, pi, jnp.where(eq, jnp.minimum(idx, pi), idx))
        val = jnp.where(gt, pv, val)
    return val, idx


def _tree_min(v, lanes):
    for sh in (8, 4, 2, 1):
        v = jnp.minimum(v, _shuf(v, lanes, sh))
    return v


def _nms_body(x1h, y1h, x2h, y2h, sh, outh,
              x1v, y1v, x2v, y2v, sv, alive, taken, row_v, rb_v, out_v, shared):
    c = lax.axis_index("c")
    s = lax.axis_index("s")
    base = s * _PER_TILE

    pltpu.sync_copy(x1h.at[pl.ds(base, _PER_TILE)], x1v)
    pltpu.sync_copy(y1h.at[pl.ds(base, _PER_TILE)], y1v)
    pltpu.sync_copy(x2h.at[pl.ds(base, _PER_TILE)], x2v)
    pltpu.sync_copy(y2h.at[pl.ds(base, _PER_TILE)], y2v)
    pltpu.sync_copy(sh.at[pl.ds(base, _PER_TILE)], sv)

    ones = jnp.ones((_L,), jnp.float32)
    zeros = jnp.zeros((_L,), jnp.float32)
    lanes = lax.iota(jnp.int32, _L)

    def init_j(j, carry):
        alive[pl.ds(j * _L, _L)] = ones
        taken[pl.ds(j * _L, _L)] = zeros
        return carry

    lax.fori_loop(0, _NVEC, init_j, 0)

    def local_pass(winner):
        bestv = jnp.full((_L,), -3e9, jnp.float32)
        besti = jnp.zeros((_L,), jnp.int32)
        for j in range(_NVEC):
            sl = pl.ds(j * _L, _L)
            svv = sv[sl]
            al = alive[sl]
            tv = taken[sl]
            if winner is None:
                anew = al
            else:
                wx1g, wy1g, wx2g, wy2g, areaw = winner
                ax1 = x1v[sl]
                ay1 = y1v[sl]
                ax2 = x2v[sl]
                ay2 = y2v[sl]
                ltx = jnp.maximum(wx1g, ax1)
                lty = jnp.maximum(wy1g, ay1)
                rbx = jnp.minimum(wx2g, ax2)
                rby = jnp.minimum(wy2g, ay2)
                inter = jnp.maximum(rbx - ltx, 0.0) * jnp.maximum(rby - lty, 0.0)
                areaa = (ax2 - ax1) * (ay2 - ay1)
                iou = inter / (areaa + areaw - inter + 1e-9)
                anew = jnp.where(iou > _NMS_THRESH, 0.0, al)
                alive[sl] = anew
            key = jnp.where((anew > 0.5) & (svv > _SCORE_THRESH), svv + 2.0,
                            jnp.where(tv > 0.5, jnp.float32(-1e9), svv))
            gidx = base + j * _L + lanes
            upd = key > bestv
            bestv = jnp.where(upd, key, bestv)
            besti = jnp.where(upd, gidx, besti)
        return bestv, besti

    bestv0, besti0 = local_pass(None)

    def round_body(k, carry):
        bestv, besti = carry
        kmaxv, gminv = _tree_argmax(bestv, besti, lanes)
        lidxv = jnp.clip(gminv - base, 0, _PER_TILE - 1)
        wx1 = plsc.load_gather(x1v, [lidxv])
        wy1 = plsc.load_gather(y1v, [lidxv])
        wx2 = plsc.load_gather(x2v, [lidxv])
        wy2 = plsc.load_gather(y2v, [lidxv])
        wsc = plsc.load_gather(sv, [lidxv])
        row = jnp.where(lanes == 0, kmaxv,
              jnp.where(lanes == 1, gminv.astype(jnp.float32),
              jnp.where(lanes == 2, wx1,
              jnp.where(lanes == 3, wy1,
              jnp.where(lanes == 4, wx2,
              jnp.where(lanes == 5, wy2,
              jnp.where(lanes == 6, wsc, zeros)))))))
        row_v[...] = row
        off = (k % 2) * _XCH
        pltpu.sync_copy(row_v, shared.at[pl.ds(off + s * _L, _L)])
        plsc.subcore_barrier()
        pltpu.sync_copy(shared.at[pl.ds(off, _XCH)], rb_v)

        keys = plsc.load_gather(rb_v, [lanes * _L])
        gidxs = plsc.load_gather(rb_v, [lanes * _L + 1])
        kmgv, gmgv = _tree_argmax(keys, gidxs, lanes)
        rstarv = _tree_min(
            jnp.where((keys == kmgv) & (gidxs == gmgv), lanes,
                      jnp.int32(_TILES + 1)), lanes)
        wbv = rstarv * _L

        def bcast(o):
            return plsc.load_gather(rb_v, [wbv + o])

        wx1g = bcast(2)
        wy1g = bcast(3)
        wx2g = bcast(4)
        wy2g = bcast(5)
        wsg = bcast(6)
        keptv = kmgv > 2.0
        areaw = (wx2g - wx1g) * (wy2g - wy1g)

        ilw = gmgv.astype(jnp.int32) - base
        owner = (ilw >= 0) & (ilw < _PER_TILE) & (lanes == 0)
        ilwc = jnp.clip(ilw, 0, _PER_TILE - 1)
        plsc.store_scatter(taken, [ilwc], ones, mask=owner)
        plsc.store_scatter(alive, [ilwc], zeros, mask=owner)

        @pl.when((c == 0) & (s == 0))
        def _emit():
            sout = jnp.where(keptv, wsg, jnp.full((_L,), -1.0, jnp.float32))
            orow = jnp.where(lanes == 0, wx1g,
                   jnp.where(lanes == 1, wy1g,
                   jnp.where(lanes == 2, wx2g,
                   jnp.where(lanes == 3, wy2g,
                   jnp.where(lanes == 4, sout, zeros)))))
            plsc.store_scatter(out_v, [k * _L + lanes], orow)

        return local_pass((wx1g, wy1g, wx2g, wy2g, areaw))

    lax.fori_loop(0, _DET, round_body, (bestv0, besti0))

    @pl.when((c == 0) & (s == 0))
    def _flush():
        pltpu.sync_copy(out_v, outh)


_nms_call = pl.kernel(
    _nms_body,
    out_type=jax.ShapeDtypeStruct((_DET * _L,), jnp.float32),
    mesh=plsc.VectorSubcoreMesh(core_axis_name="c", subcore_axis_name="s"),
    compiler_params=pltpu.CompilerParams(needs_layout_passes=False),
    scratch_types=[
        pltpu.VMEM((_PER_TILE,), jnp.float32),
        pltpu.VMEM((_PER_TILE,), jnp.float32),
        pltpu.VMEM((_PER_TILE,), jnp.float32),
        pltpu.VMEM((_PER_TILE,), jnp.float32),
        pltpu.VMEM((_PER_TILE,), jnp.float32),
        pltpu.VMEM((_PER_TILE,), jnp.float32),
        pltpu.VMEM((_PER_TILE,), jnp.float32),
        pltpu.VMEM((_L,), jnp.float32),
        pltpu.VMEM((_XCH,), jnp.float32),
        pltpu.VMEM((_DET * _L,), jnp.float32),
        pltpu.VMEM_SHARED((2 * _XCH,), jnp.float32),
    ],
)


@jax.jit
def kernel(boxes, scores):
    pad = _NPAD - _N
    x1 = jnp.pad(boxes[:, 0], (0, pad))
    y1 = jnp.pad(boxes[:, 1], (0, pad))
    x2 = jnp.pad(boxes[:, 2], (0, pad))
    y2 = jnp.pad(boxes[:, 3], (0, pad))
    sp = jnp.pad(scores, (0, pad), constant_values=-1.0)
    out = _nms_call(x1, y1, x2, y2, sp)
    return out.reshape(_DET, _L)[:, :5]
